# baseline (speedup 1.0000x reference)
import functools

import jax
import jax.numpy as jnp
from jax import lax
from jax.experimental import pallas as pl
from jax.experimental.pallas import tpu as pltpu
from jax.experimental.pallas import tpu_sc as plsc

POSITIONS = 200
BATCH = 16384
FLAT = POSITIONS * BATCH
NCORES = 2
NSUB = 16
LANES = 16

PER_WORKER = FLAT // (NCORES * NSUB)
ZBUF = 12800
NZDMA = PER_WORKER // ZBUF
CHUNK = BATCH // NSUB
NIDX = CHUNK // 128


def _sc_onehot(inp_hbm, out_hbm, zbuf, idx_v, off_v, ones_v, zsem, ssem):
    cid = lax.axis_index("c")
    sid = lax.axis_index("s")
    region = (cid * NSUB + sid) * PER_WORKER

    def _zero_body(i, carry):
        zbuf[pl.ds(i * LANES, LANES)] = jnp.zeros((LANES,), jnp.float32)
        return carry

    lax.fori_loop(0, ZBUF // LANES, _zero_body, 0)

    zcopies = [
        pltpu.async_copy(zbuf, out_hbm.at[pl.ds(region + d * ZBUF, ZBUF)], zsem)
        for d in range(NZDMA)
    ]

    pltpu.sync_copy(inp_hbm.at[pl.ds(sid * CHUNK, CHUNK)], idx_v)
    iota = lax.broadcasted_iota(jnp.int32, (LANES,), 0)
    base = sid * CHUNK
    for k in range(CHUNK // LANES):
        vals = idx_v[pl.ds(k * LANES, LANES)]
        off = vals * BATCH + (base + k * LANES + iota)
        off_v[k // 8, pl.ds((k % 8) * LANES, LANES)] = off
    for m in range(128 // LANES):
        ones_v[pl.ds(m * LANES, LANES)] = jnp.full((LANES,), 1.0, jnp.float32)

    for c in zcopies:
        c.wait()
    plsc.subcore_barrier()

    scopies = [
        pltpu.async_copy(ones_v, out_hbm.at[off_v.at[j]], ssem)
        for j in range(NIDX)
    ]
    for c in scopies:
        c.wait()


@functools.partial(jax.jit, donate_argnums=())
def kernel(inputs):
    k = functools.partial(
        pl.kernel,
        mesh=plsc.VectorSubcoreMesh(core_axis_name="c", subcore_axis_name="s"),
        out_type=jax.ShapeDtypeStruct((FLAT,), jnp.float32),
        scratch_types=[
            pltpu.VMEM((ZBUF,), jnp.float32),
            pltpu.VMEM((CHUNK,), jnp.int32),
            pltpu.VMEM((NIDX, 128), jnp.int32),
            pltpu.VMEM((128,), jnp.float32),
            pltpu.SemaphoreType.DMA,
            pltpu.SemaphoreType.DMA,
        ],
    )(_sc_onehot)
    out_flat = k(inputs)
    return out_flat.reshape(POSITIONS, BATCH).T


# speedup vs baseline: 13.5898x; 13.5898x over previous
import jax
import jax.numpy as jnp
from jax.experimental import pallas as pl
from jax.experimental.pallas import tpu as pltpu

POSITIONS = 200
CHUNK = 4096
NCHUNK = 4


def _onehot_t_block(in_ref, out_ref):
    vals = in_ref[0, 0, :]
    rows = jax.lax.broadcasted_iota(jnp.int32, (POSITIONS, CHUNK), 0)
    out_ref[:, :] = (vals[None, :] == rows).astype(jnp.float32)


def kernel(inputs):
    n = inputs.shape[0]
    inputs3 = inputs.reshape(NCHUNK, 1, CHUNK)
    out_t = pl.pallas_call(
        _onehot_t_block,
        grid=(NCHUNK,),
        in_specs=[pl.BlockSpec((1, 1, CHUNK), lambda i: (i, 0, 0))],
        out_specs=pl.BlockSpec((POSITIONS, CHUNK), lambda i: (0, i)),
        out_shape=jax.ShapeDtypeStruct((POSITIONS, n), jnp.float32),
        compiler_params=pltpu.CompilerParams(
            dimension_semantics=("parallel",),
        ),
    )(inputs3)
    return out_t.T
